# baseline (speedup 1.0000x reference)
import functools

import jax
import jax.numpy as jnp
from jax import lax
from jax.experimental import pallas as pl
from jax.experimental.pallas import tpu as pltpu
from jax.experimental.pallas import tpu_sc as plsc

NE_ = 8192
D_ = 256
B_ = 8
T_ = 1024
BETA_ = 0.25

BN_ = 2048
NB_ = NE_ // BN_


def _argmin_body(z_ref, emb_ref, idx_ref, rows_ref, e2_ref, es_ref):
    @pl.when(pl.program_id(0) == 0)
    def _():
        rows_ref[...] = lax.broadcasted_iota(
            jnp.int32, (BN_, T_), 0).astype(jnp.float32)
        es_ref[...] = emb_ref[...] * jnp.float32(-2.0)
        for n in range(NB_):
            es = es_ref[n * BN_:(n + 1) * BN_, :]
            e2_ref[:, n:n + 1] = 0.25 * jnp.sum(es * es, axis=1, keepdims=True)

    zb = z_ref[0]
    rows = rows_ref[...]
    best = None
    besti = None
    for n in range(NB_):
        es = es_ref[n * BN_:(n + 1) * BN_, :]
        d = e2_ref[:, n:n + 1] + jnp.dot(es, zb, preferred_element_type=jnp.float32)
        lmin = jnp.min(d, axis=0, keepdims=True)
        larg = jnp.min(jnp.where(d == lmin, rows, jnp.float32(2.0**30)),
                       axis=0, keepdims=True) + jnp.float32(n * BN_)
        if n == 0:
            best, besti = lmin, larg
        else:
            take = lmin < best
            besti = jnp.where(take, larg, besti)
            best = jnp.where(take, lmin, best)
    idx_ref[0] = besti.astype(jnp.int32)


def _argmin_call(zr, emb):
    return pl.pallas_call(
        _argmin_body,
        grid=(B_,),
        in_specs=[
            pl.BlockSpec((1, D_, T_), lambda b: (b, 0, 0)),
            pl.BlockSpec((NE_, D_), lambda b: (0, 0)),
        ],
        out_specs=pl.BlockSpec((1, 1, T_), lambda b: (b, 0, 0)),
        out_shape=jax.ShapeDtypeStruct((B_, 1, T_), jnp.int32),
        scratch_shapes=[
            pltpu.VMEM((BN_, T_), jnp.float32),
            pltpu.VMEM((BN_, NB_), jnp.float32),
            pltpu.VMEM((NE_, D_), jnp.float32),
        ],
    )(zr, emb)


def _finish_body(zq_ref, z_ref, out_ref, loss_ref):
    b = pl.program_id(0)
    zqt = zq_ref[0].T
    zb = z_ref[0]
    dif = zqt - zb
    out_ref[0] = zb + dif
    s = jnp.sum(dif * dif).reshape(1, 1)

    @pl.when(b == 0)
    def _():
        loss_ref[...] = s

    @pl.when(b > 0)
    def _():
        loss_ref[...] += s


def _finish_call(zq, zr):
    return pl.pallas_call(
        _finish_body,
        grid=(B_,),
        in_specs=[
            pl.BlockSpec((1, T_, D_), lambda b: (b, 0, 0)),
            pl.BlockSpec((1, D_, T_), lambda b: (b, 0, 0)),
        ],
        out_specs=[
            pl.BlockSpec((1, D_, T_), lambda b: (b, 0, 0)),
            pl.BlockSpec((1, 1), lambda b: (0, 0)),
        ],
        out_shape=[
            jax.ShapeDtypeStruct((B_, D_, T_), jnp.float32),
            jax.ShapeDtypeStruct((1, 1), jnp.float32),
        ],
    )(zq, zr)


_NC = 2
_NS = 16
_NW = _NC * _NS
_TT = B_ * T_
_BPW = _TT // _NW
_CH = 128
_NCH = _BPW // _CH


def _gather_call(emb, idx2):
    mesh = plsc.VectorSubcoreMesh(core_axis_name="c", subcore_axis_name="s")

    @functools.partial(
        pl.kernel,
        mesh=mesh,
        out_type=jax.ShapeDtypeStruct((_TT, D_), jnp.float32),
        scratch_types=[
            pltpu.VMEM((_NCH, _CH), jnp.int32),
            pltpu.VMEM((_BPW, D_), jnp.float32),
            pltpu.SemaphoreType.DMA,
        ],
    )
    def gather_k(emb_hbm, idx_hbm, out_hbm, idx_v, rows_v, sem):
        wid = lax.axis_index("s") * _NC + lax.axis_index("c")
        pltpu.sync_copy(idx_hbm.at[pl.ds(wid * _NCH, _NCH)], idx_v)
        cps = [
            pltpu.async_copy(emb_hbm.at[idx_v.at[j]],
                             rows_v.at[pl.ds(j * _CH, _CH)], sem)
            for j in range(_NCH)
        ]
        for cp in cps:
            cp.wait()
        pltpu.sync_copy(rows_v, out_hbm.at[pl.ds(wid * _BPW, _BPW)])

    return gather_k(emb, idx2)


def kernel(z, emb):
    zr = z.reshape(B_, D_, T_)
    idx3 = _argmin_call(zr, emb)
    zq = _gather_call(emb, idx3.reshape(_NW * _NCH, _CH))
    zqt, loss_sum = _finish_call(zq.reshape(B_, T_, D_), zr)
    z_q_out = zqt.reshape(B_, D_, 32, 32)
    index = idx3.reshape(B_, 32, 32)
    loss = loss_sum[0, 0] * ((1.0 + BETA_) / (B_ * T_ * D_))
    return z_q_out, index, loss


# speedup vs baseline: 1.0450x; 1.0450x over previous
import functools

import jax
import jax.numpy as jnp
from jax import lax
from jax.experimental import pallas as pl
from jax.experimental.pallas import tpu as pltpu
from jax.experimental.pallas import tpu_sc as plsc

NE_ = 8192
D_ = 256
B_ = 8
T_ = 1024
BETA_ = 0.25

BN_ = 1024
NB_ = NE_ // BN_


def _argmin_body(z_ref, emb_ref, idx_ref, loss_ref, rows_ref, e2_ref, es_ref):
    @pl.when(pl.program_id(0) == 0)
    def _():
        rows_ref[...] = lax.broadcasted_iota(
            jnp.int32, (BN_, T_), 0).astype(jnp.float32)
        es_ref[...] = emb_ref[...] * jnp.float32(-2.0)
        for n in range(NB_):
            es = es_ref[n * BN_:(n + 1) * BN_, :]
            e2_ref[:, n:n + 1] = 0.25 * jnp.sum(es * es, axis=1, keepdims=True)

    zb = z_ref[0]
    rows = rows_ref[...]
    best = None
    besti = None
    for n in range(NB_):
        es = es_ref[n * BN_:(n + 1) * BN_, :]
        d = e2_ref[:, n:n + 1] + jnp.dot(es, zb, preferred_element_type=jnp.float32)
        lmin = jnp.min(d, axis=0, keepdims=True)
        larg = jnp.min(jnp.where(d == lmin, rows, jnp.float32(2.0**30)),
                       axis=0, keepdims=True) + jnp.float32(n * BN_)
        if n == 0:
            best, besti = lmin, larg
        else:
            take = lmin < best
            besti = jnp.where(take, larg, besti)
            best = jnp.where(take, lmin, best)
    idx_ref[0] = besti.astype(jnp.int32)

    s = (jnp.sum(best) + jnp.sum(zb * zb)).reshape(1, 1)

    @pl.when(pl.program_id(0) == 0)
    def _():
        loss_ref[...] = s

    @pl.when(pl.program_id(0) > 0)
    def _():
        loss_ref[...] += s


def _argmin_call(zr, emb):
    return pl.pallas_call(
        _argmin_body,
        grid=(B_,),
        in_specs=[
            pl.BlockSpec((1, D_, T_), lambda b: (b, 0, 0)),
            pl.BlockSpec((NE_, D_), lambda b: (0, 0)),
        ],
        out_specs=[
            pl.BlockSpec((1, 1, T_), lambda b: (b, 0, 0)),
            pl.BlockSpec((1, 1), lambda b: (0, 0)),
        ],
        out_shape=[
            jax.ShapeDtypeStruct((B_, 1, T_), jnp.int32),
            jax.ShapeDtypeStruct((1, 1), jnp.float32),
        ],
        scratch_shapes=[
            pltpu.VMEM((BN_, T_), jnp.float32),
            pltpu.VMEM((BN_, NB_), jnp.float32),
            pltpu.VMEM((NE_, D_), jnp.float32),
        ],
    )(zr, emb)


def _finish_body(zq_ref, out_ref):
    out_ref[0] = zq_ref[0].T


def _finish_call(zq):
    return pl.pallas_call(
        _finish_body,
        grid=(B_,),
        in_specs=[
            pl.BlockSpec((1, T_, D_), lambda b: (b, 0, 0)),
        ],
        out_specs=pl.BlockSpec((1, D_, T_), lambda b: (b, 0, 0)),
        out_shape=jax.ShapeDtypeStruct((B_, D_, T_), jnp.float32),
    )(zq)


_NC = 2
_NS = 16
_NW = _NC * _NS
_TT = B_ * T_
_BPW = _TT // _NW
_CH = 128
_NCH = _BPW // _CH


def _gather_call(emb, idx2):
    mesh = plsc.VectorSubcoreMesh(core_axis_name="c", subcore_axis_name="s")

    @functools.partial(
        pl.kernel,
        mesh=mesh,
        out_type=jax.ShapeDtypeStruct((_TT, D_), jnp.float32),
        scratch_types=[
            pltpu.VMEM((_NCH, _CH), jnp.int32),
            pltpu.VMEM((_BPW, D_), jnp.float32),
            pltpu.SemaphoreType.DMA,
        ],
    )
    def gather_k(emb_hbm, idx_hbm, out_hbm, idx_v, rows_v, sem):
        wid = lax.axis_index("s") * _NC + lax.axis_index("c")
        pltpu.sync_copy(idx_hbm.at[pl.ds(wid * _NCH, _NCH)], idx_v)
        cps = [
            pltpu.async_copy(emb_hbm.at[idx_v.at[j]],
                             rows_v.at[pl.ds(j * _CH, _CH)], sem)
            for j in range(_NCH)
        ]
        for cp in cps:
            cp.wait()
        pltpu.sync_copy(rows_v, out_hbm.at[pl.ds(wid * _BPW, _BPW)])

    return gather_k(emb, idx2)


def kernel(z, emb):
    zr = z.reshape(B_, D_, T_)
    idx3, loss_sum = _argmin_call(zr, emb)
    zq = _gather_call(emb, idx3.reshape(_NW * _NCH, _CH))
    zqt = _finish_call(zq.reshape(B_, T_, D_))
    z_q_out = zqt.reshape(B_, D_, 32, 32)
    index = idx3.reshape(B_, 32, 32)
    loss = loss_sum[0, 0] * ((1.0 + BETA_) / (B_ * T_ * D_))
    return z_q_out, index, loss


# speedup vs baseline: 1.2358x; 1.1826x over previous
import functools

import jax
import jax.numpy as jnp
from jax import lax
from jax.experimental import pallas as pl
from jax.experimental.pallas import tpu as pltpu
from jax.experimental.pallas import tpu_sc as plsc

NE_ = 8192
D_ = 256
B_ = 8
T_ = 1024
BETA_ = 0.25

BN_ = 1024
NB_ = NE_ // BN_


def _argmin_body(z_ref, emb_ref, idx_ref, rows_ref, e2_ref, es_ref):
    @pl.when(pl.program_id(0) == 0)
    def _():
        rows_ref[...] = lax.broadcasted_iota(
            jnp.int32, (BN_, T_), 0).astype(jnp.float32)
        es_ref[...] = emb_ref[...] * jnp.float32(-2.0)
        for n in range(NB_):
            es = es_ref[n * BN_:(n + 1) * BN_, :]
            e2_ref[:, n:n + 1] = 0.25 * jnp.sum(es * es, axis=1, keepdims=True)

    zb = z_ref[0]
    rows = rows_ref[...]
    best = None
    besti = None
    for n in range(NB_):
        es = es_ref[n * BN_:(n + 1) * BN_, :]
        d = e2_ref[:, n:n + 1] + jnp.dot(es, zb, preferred_element_type=jnp.float32)
        lmin = jnp.min(d, axis=0, keepdims=True)
        larg = (jnp.argmin(d, axis=0).astype(jnp.float32)
                + jnp.float32(n * BN_))[None, :]
        if n == 0:
            best, besti = lmin, larg
        else:
            take = lmin < best
            besti = jnp.where(take, larg, besti)
            best = jnp.where(take, lmin, best)
    idx_ref[0] = besti.astype(jnp.int32)


def _argmin_call(zr, emb):
    return pl.pallas_call(
        _argmin_body,
        grid=(B_,),
        in_specs=[
            pl.BlockSpec((1, D_, T_), lambda b: (b, 0, 0)),
            pl.BlockSpec((NE_, D_), lambda b: (0, 0)),
        ],
        out_specs=pl.BlockSpec((1, 1, T_), lambda b: (b, 0, 0)),
        out_shape=jax.ShapeDtypeStruct((B_, 1, T_), jnp.int32),
        scratch_shapes=[
            pltpu.VMEM((BN_, T_), jnp.float32),
            pltpu.VMEM((BN_, NB_), jnp.float32),
            pltpu.VMEM((NE_, D_), jnp.float32),
        ],
    )(zr, emb)


def _finish_body(zq_ref, z_ref, out_ref, loss_ref):
    b = pl.program_id(0)
    zqt = zq_ref[0].T
    zb = z_ref[0]
    dif = zqt - zb
    out_ref[0] = zb + dif
    s = jnp.sum(dif * dif).reshape(1, 1)

    @pl.when(b == 0)
    def _():
        loss_ref[...] = s

    @pl.when(b > 0)
    def _():
        loss_ref[...] += s


def _finish_call(zq, zr):
    return pl.pallas_call(
        _finish_body,
        grid=(B_,),
        in_specs=[
            pl.BlockSpec((1, T_, D_), lambda b: (b, 0, 0)),
            pl.BlockSpec((1, D_, T_), lambda b: (b, 0, 0)),
        ],
        out_specs=[
            pl.BlockSpec((1, D_, T_), lambda b: (b, 0, 0)),
            pl.BlockSpec((1, 1), lambda b: (0, 0)),
        ],
        out_shape=[
            jax.ShapeDtypeStruct((B_, D_, T_), jnp.float32),
            jax.ShapeDtypeStruct((1, 1), jnp.float32),
        ],
    )(zq, zr)


_NC = 2
_NS = 16
_NW = _NC * _NS
_TT = B_ * T_
_BPW = _TT // _NW
_CH = 128
_NCH = _BPW // _CH


def _gather_call(emb, idx2):
    mesh = plsc.VectorSubcoreMesh(core_axis_name="c", subcore_axis_name="s")

    @functools.partial(
        pl.kernel,
        mesh=mesh,
        out_type=jax.ShapeDtypeStruct((_TT, D_), jnp.float32),
        scratch_types=[
            pltpu.VMEM((_NCH, _CH), jnp.int32),
            pltpu.VMEM((_BPW, D_), jnp.float32),
            pltpu.SemaphoreType.DMA,
        ],
    )
    def gather_k(emb_hbm, idx_hbm, out_hbm, idx_v, rows_v, sem):
        wid = lax.axis_index("s") * _NC + lax.axis_index("c")
        pltpu.sync_copy(idx_hbm.at[pl.ds(wid * _NCH, _NCH)], idx_v)
        cps = [
            pltpu.async_copy(emb_hbm.at[idx_v.at[j]],
                             rows_v.at[pl.ds(j * _CH, _CH)], sem)
            for j in range(_NCH)
        ]
        for cp in cps:
            cp.wait()
        pltpu.sync_copy(rows_v, out_hbm.at[pl.ds(wid * _BPW, _BPW)])

    return gather_k(emb, idx2)


def kernel(z, emb):
    zr = z.reshape(B_, D_, T_)
    idx3 = _argmin_call(zr, emb)
    zq = _gather_call(emb, idx3.reshape(_NW * _NCH, _CH))
    zqt, loss_sum = _finish_call(zq.reshape(B_, T_, D_), zr)
    z_q_out = zqt.reshape(B_, D_, 32, 32)
    index = idx3.reshape(B_, 32, 32)
    loss = loss_sum[0, 0] * ((1.0 + BETA_) / (B_ * T_ * D_))
    return z_q_out, index, loss


# speedup vs baseline: 1.2360x; 1.0002x over previous
import functools

import jax
import jax.numpy as jnp
from jax import lax
from jax.experimental import pallas as pl
from jax.experimental.pallas import tpu as pltpu
from jax.experimental.pallas import tpu_sc as plsc

NE_ = 8192
D_ = 256
B_ = 8
T_ = 1024
BETA_ = 0.25

BN_ = 1024
NB_ = NE_ // BN_


def _argmin_body(z_ref, emb_ref, idx_ref, e2_ref, es_ref):
    @pl.when(pl.program_id(0) == 0)
    def _():
        es_ref[...] = emb_ref[...] * jnp.float32(-2.0)
        for n in range(NB_):
            es = es_ref[n * BN_:(n + 1) * BN_, :]
            e2_ref[:, n:n + 1] = 0.25 * jnp.sum(es * es, axis=1, keepdims=True)

    zb = z_ref[0]
    best = None
    besti = None
    for n in range(NB_):
        es = es_ref[n * BN_:(n + 1) * BN_, :]
        d = e2_ref[:, n:n + 1] + jnp.dot(es, zb, preferred_element_type=jnp.float32)
        lmin = jnp.min(d, axis=0, keepdims=True)
        larg = (jnp.argmin(d, axis=0).astype(jnp.float32)
                + jnp.float32(n * BN_))[None, :]
        if n == 0:
            best, besti = lmin, larg
        else:
            take = lmin < best
            besti = jnp.where(take, larg, besti)
            best = jnp.where(take, lmin, best)
    idx_ref[0] = besti.astype(jnp.int32)


def _argmin_call(zr, emb):
    return pl.pallas_call(
        _argmin_body,
        grid=(B_,),
        in_specs=[
            pl.BlockSpec((1, D_, T_), lambda b: (b, 0, 0)),
            pl.BlockSpec((NE_, D_), lambda b: (0, 0)),
        ],
        out_specs=pl.BlockSpec((1, 1, T_), lambda b: (b, 0, 0)),
        out_shape=jax.ShapeDtypeStruct((B_, 1, T_), jnp.int32),
        scratch_shapes=[
            pltpu.VMEM((BN_, NB_), jnp.float32),
            pltpu.VMEM((NE_, D_), jnp.float32),
        ],
    )(zr, emb)


def _finish_body(zq_ref, z_ref, out_ref, loss_ref):
    b = pl.program_id(0)
    zqt = zq_ref[0].T
    zb = z_ref[0]
    dif = zqt - zb
    out_ref[0] = zb + dif
    s = jnp.sum(dif * dif).reshape(1, 1)

    @pl.when(b == 0)
    def _():
        loss_ref[...] = s

    @pl.when(b > 0)
    def _():
        loss_ref[...] += s


def _finish_call(zq, zr):
    return pl.pallas_call(
        _finish_body,
        grid=(B_,),
        in_specs=[
            pl.BlockSpec((1, T_, D_), lambda b: (b, 0, 0)),
            pl.BlockSpec((1, D_, T_), lambda b: (b, 0, 0)),
        ],
        out_specs=[
            pl.BlockSpec((1, D_, T_), lambda b: (b, 0, 0)),
            pl.BlockSpec((1, 1), lambda b: (0, 0)),
        ],
        out_shape=[
            jax.ShapeDtypeStruct((B_, D_, T_), jnp.float32),
            jax.ShapeDtypeStruct((1, 1), jnp.float32),
        ],
    )(zq, zr)


_NC = 2
_NS = 16
_NW = _NC * _NS
_TT = B_ * T_
_BPW = _TT // _NW
_CH = 128
_NCH = _BPW // _CH


def _gather_call(emb, idx2):
    mesh = plsc.VectorSubcoreMesh(core_axis_name="c", subcore_axis_name="s")

    @functools.partial(
        pl.kernel,
        mesh=mesh,
        out_type=jax.ShapeDtypeStruct((_TT, D_), jnp.float32),
        scratch_types=[
            pltpu.VMEM((_NCH, _CH), jnp.int32),
            pltpu.VMEM((_BPW, D_), jnp.float32),
            pltpu.SemaphoreType.DMA,
        ],
    )
    def gather_k(emb_hbm, idx_hbm, out_hbm, idx_v, rows_v, sem):
        wid = lax.axis_index("s") * _NC + lax.axis_index("c")
        pltpu.sync_copy(idx_hbm.at[pl.ds(wid * _NCH, _NCH)], idx_v)
        cps = [
            pltpu.async_copy(emb_hbm.at[idx_v.at[j]],
                             rows_v.at[pl.ds(j * _CH, _CH)], sem)
            for j in range(_NCH)
        ]
        for cp in cps:
            cp.wait()
        pltpu.sync_copy(rows_v, out_hbm.at[pl.ds(wid * _BPW, _BPW)])

    return gather_k(emb, idx2)


def kernel(z, emb):
    zr = z.reshape(B_, D_, T_)
    idx3 = _argmin_call(zr, emb)
    zq = _gather_call(emb, idx3.reshape(_NW * _NCH, _CH))
    zqt, loss_sum = _finish_call(zq.reshape(B_, T_, D_), zr)
    z_q_out = zqt.reshape(B_, D_, 32, 32)
    index = idx3.reshape(B_, 32, 32)
    loss = loss_sum[0, 0] * ((1.0 + BETA_) / (B_ * T_ * D_))
    return z_q_out, index, loss


# speedup vs baseline: 1.2488x; 1.0104x over previous
import functools

import jax
import jax.numpy as jnp
from jax import lax
from jax.experimental import pallas as pl
from jax.experimental.pallas import tpu as pltpu
from jax.experimental.pallas import tpu_sc as plsc

NE_ = 8192
D_ = 256
B_ = 8
T_ = 1024
BETA_ = 0.25

BN_ = 512
NB_ = NE_ // BN_


def _argmin_body(z_ref, emb_ref, idx_ref, e2_ref, es_ref):
    @pl.when(pl.program_id(0) == 0)
    def _():
        es_ref[...] = emb_ref[...] * jnp.float32(-2.0)
        for n in range(NB_):
            es = es_ref[n * BN_:(n + 1) * BN_, :]
            e2_ref[:, n:n + 1] = 0.25 * jnp.sum(es * es, axis=1, keepdims=True)

    zb = z_ref[0]
    best = None
    besti = None
    for n in range(NB_):
        es = es_ref[n * BN_:(n + 1) * BN_, :]
        d = e2_ref[:, n:n + 1] + jnp.dot(es, zb, preferred_element_type=jnp.float32)
        lmin = jnp.min(d, axis=0, keepdims=True)
        larg = (jnp.argmin(d, axis=0).astype(jnp.float32)
                + jnp.float32(n * BN_))[None, :]
        if n == 0:
            best, besti = lmin, larg
        else:
            take = lmin < best
            besti = jnp.where(take, larg, besti)
            best = jnp.where(take, lmin, best)
    idx_ref[0] = besti.astype(jnp.int32)


def _argmin_call(zr, emb):
    return pl.pallas_call(
        _argmin_body,
        grid=(B_,),
        in_specs=[
            pl.BlockSpec((1, D_, T_), lambda b: (b, 0, 0)),
            pl.BlockSpec((NE_, D_), lambda b: (0, 0)),
        ],
        out_specs=pl.BlockSpec((1, 1, T_), lambda b: (b, 0, 0)),
        out_shape=jax.ShapeDtypeStruct((B_, 1, T_), jnp.int32),
        scratch_shapes=[
            pltpu.VMEM((BN_, NB_), jnp.float32),
            pltpu.VMEM((NE_, D_), jnp.float32),
        ],
    )(zr, emb)


def _finish_body(zq_ref, z_ref, out_ref, loss_ref):
    b = pl.program_id(0)
    zqt = zq_ref[0].T
    zb = z_ref[0]
    dif = zqt - zb
    out_ref[0] = zb + dif
    s = jnp.sum(dif * dif).reshape(1, 1)

    @pl.when(b == 0)
    def _():
        loss_ref[...] = s

    @pl.when(b > 0)
    def _():
        loss_ref[...] += s


def _finish_call(zq, zr):
    return pl.pallas_call(
        _finish_body,
        grid=(B_,),
        in_specs=[
            pl.BlockSpec((1, T_, D_), lambda b: (b, 0, 0)),
            pl.BlockSpec((1, D_, T_), lambda b: (b, 0, 0)),
        ],
        out_specs=[
            pl.BlockSpec((1, D_, T_), lambda b: (b, 0, 0)),
            pl.BlockSpec((1, 1), lambda b: (0, 0)),
        ],
        out_shape=[
            jax.ShapeDtypeStruct((B_, D_, T_), jnp.float32),
            jax.ShapeDtypeStruct((1, 1), jnp.float32),
        ],
    )(zq, zr)


_NC = 2
_NS = 16
_NW = _NC * _NS
_TT = B_ * T_
_BPW = _TT // _NW
_CH = 128
_NCH = _BPW // _CH


def _gather_call(emb, idx2):
    mesh = plsc.VectorSubcoreMesh(core_axis_name="c", subcore_axis_name="s")

    @functools.partial(
        pl.kernel,
        mesh=mesh,
        out_type=jax.ShapeDtypeStruct((_TT, D_), jnp.float32),
        scratch_types=[
            pltpu.VMEM((_NCH, _CH), jnp.int32),
            pltpu.VMEM((_BPW, D_), jnp.float32),
            pltpu.SemaphoreType.DMA,
        ],
    )
    def gather_k(emb_hbm, idx_hbm, out_hbm, idx_v, rows_v, sem):
        wid = lax.axis_index("s") * _NC + lax.axis_index("c")
        pltpu.sync_copy(idx_hbm.at[pl.ds(wid * _NCH, _NCH)], idx_v)
        cps = [
            pltpu.async_copy(emb_hbm.at[idx_v.at[j]],
                             rows_v.at[pl.ds(j * _CH, _CH)], sem)
            for j in range(_NCH)
        ]
        for cp in cps:
            cp.wait()
        pltpu.sync_copy(rows_v, out_hbm.at[pl.ds(wid * _BPW, _BPW)])

    return gather_k(emb, idx2)


def kernel(z, emb):
    zr = z.reshape(B_, D_, T_)
    idx3 = _argmin_call(zr, emb)
    zq = _gather_call(emb, idx3.reshape(_NW * _NCH, _CH))
    zqt, loss_sum = _finish_call(zq.reshape(B_, T_, D_), zr)
    z_q_out = zqt.reshape(B_, D_, 32, 32)
    index = idx3.reshape(B_, 32, 32)
    loss = loss_sum[0, 0] * ((1.0 + BETA_) / (B_ * T_ * D_))
    return z_q_out, index, loss
